# baseline (speedup 1.0000x reference)
import jax
import jax.numpy as jnp
from jax import lax
from jax.experimental import pallas as pl
from jax.experimental.pallas import tpu as pltpu
from jax.experimental.pallas import tpu_sc as plsc

G = 4
N = 4096
Z = 128
C = 8
CG = G * C
NC = 2
NS = 16
NW = NC * NS
ROWS = G * N
RPW = ROWS // NW
SCH = 128
NSCH = RPW // SCH
GCH = 128
NGCH = RPW // GCH
LANES = 16
FB = Z // LANES
IDXR = 128

_mesh = plsc.VectorSubcoreMesh(core_axis_name="c", subcore_axis_name="s")


def _seg_sum_body(x_hbm, idx_hbm, out_hbm, xb0, xb1, xb2, xb3, idxbuf, zbuf,
                  acc_sh, sem0, sem1, sem2, sem3, ssem):
    c = lax.axis_index("c")
    s = lax.axis_index("s")
    w = s * NC + c
    base = w * RPW

    xbufs = (xb0, xb1, xb2, xb3)
    sems = (sem0, sem1, sem2, sem3)
    cps = [
        pltpu.async_copy(x_hbm.at[pl.ds(base + j * SCH, SCH)], xbufs[j], sems[j])
        for j in range(NSCH)
    ]
    pltpu.sync_copy(idx_hbm.at[pl.ds(w * (RPW // IDXR), RPW // IDXR)], idxbuf)

    zero = jnp.zeros((LANES,), jnp.float32)
    for r in range(CG):
        for f in range(FB):
            zbuf[r, pl.ds(f * LANES, LANES)] = zero

    @pl.when(s == 0)
    def _():
        pltpu.sync_copy(zbuf, acc_sh)

    plsc.subcore_barrier()

    scs = []
    for j in range(NSCH):
        cps[j].wait()
        scs.append(
            pltpu.async_copy(xbufs[j], acc_sh.at[idxbuf.at[j]], ssem, add=True))
    for sc in scs:
        sc.wait()

    plsc.subcore_barrier()

    @pl.when(s == 0)
    def _():
        pltpu.sync_copy(acc_sh, out_hbm.at[c])


_seg_sum = pl.kernel(
    _seg_sum_body,
    out_type=jax.ShapeDtypeStruct((NC, CG, Z), jnp.float32),
    mesh=_mesh,
    compiler_params=pltpu.CompilerParams(skip_device_barrier=True),
    scratch_types=[
        pltpu.VMEM((SCH, Z), jnp.float32),
        pltpu.VMEM((SCH, Z), jnp.float32),
        pltpu.VMEM((SCH, Z), jnp.float32),
        pltpu.VMEM((SCH, Z), jnp.float32),
        pltpu.VMEM((RPW // IDXR, IDXR), jnp.int32),
        pltpu.VMEM((CG, Z), jnp.float32),
        pltpu.VMEM_SHARED((CG, Z), jnp.float32),
        pltpu.SemaphoreType.DMA,
        pltpu.SemaphoreType.DMA,
        pltpu.SemaphoreType.DMA,
        pltpu.SemaphoreType.DMA,
        pltpu.SemaphoreType.DMA,
    ],
)


def _dense_body(sums_ref, lab_ref, w1_ref, b1_ref, w2_ref, b2_ref, out_ref):
    sums = sums_ref[0] + sums_ref[1]
    lab = lab_ref[...]
    rowc = lax.broadcasted_iota(jnp.int32, (CG, 1), 0) % C
    cnt = jnp.zeros((CG, 1), jnp.float32)
    for cc in range(C):
        n_cc = jnp.sum(jnp.where(lab == cc, 1.0, 0.0))
        cnt = jnp.where(rowc == cc, n_cc, cnt)
    m = sums / jnp.maximum(cnt, 1.0)
    h1 = jnp.dot(m, w1_ref[...], preferred_element_type=jnp.float32)
    h1 = jnp.maximum(h1 + b1_ref[...], 0.0)
    h2 = jnp.dot(h1, w2_ref[...], preferred_element_type=jnp.float32)
    out_ref[...] = h2 + b2_ref[...]


_dense = pl.pallas_call(
    _dense_body,
    out_shape=jax.ShapeDtypeStruct((CG, Z), jnp.float32),
)


def _gadd_chunk(xbuf, idxbuf, h2buf, chunk):
    lo = chunk * (GCH // LANES)

    def body(b, carry):
        jrow = b // (IDXR // LANES)
        goff = b % (IDXR // LANES)
        cv = idxbuf[jrow, pl.ds(goff * LANES, LANES)]
        i0 = (b % (GCH // LANES)) * LANES

        def loads(t):
            cls = cv[t]
            return [h2buf[cls, pl.ds(f * LANES, LANES)] for f in range(FB)]

        vs = loads(0)
        for t in range(LANES):
            nxt = loads(t + 1) if t + 1 < LANES else None
            for f in range(FB):
                plsc.addupdate(xbuf.at[i0 + t, pl.ds(f * LANES, LANES)], vs[f])
            vs = nxt
        return carry

    lax.fori_loop(lo, lo + GCH // LANES, body, 0)


def _gather_add_body(x_hbm, idx_hbm, h2_hbm, out_hbm, xb0, xb1, xb2, idxbuf,
                     h2buf, sem0, sem1, sem2, osem):
    c = lax.axis_index("c")
    s = lax.axis_index("s")
    w = s * NC + c
    base = w * RPW

    bufs = (xb0, xb1, xb2)
    sems = (sem0, sem1, sem2)

    def load(j):
        return pltpu.async_copy(
            x_hbm.at[pl.ds(base + j * GCH, GCH)], bufs[j % 3], sems[j % 3])

    cps = [load(j) for j in range(3)]
    pltpu.sync_copy(h2_hbm, h2buf)
    pltpu.sync_copy(idx_hbm.at[pl.ds(w * (RPW // IDXR), RPW // IDXR)], idxbuf)

    sts = []
    for j in range(NGCH):
        cps[j].wait()
        _gadd_chunk(bufs[j % 3], idxbuf, h2buf, j)
        sts.append(pltpu.async_copy(
            bufs[j % 3], out_hbm.at[pl.ds(base + j * GCH, GCH)], osem))
        if j + 3 < NGCH:
            sts[j].wait()
            cps.append(load(j + 3))
    for j in range(NGCH):
        if not (j + 3 < NGCH):
            sts[j].wait()


_gather_add = pl.kernel(
    _gather_add_body,
    out_type=jax.ShapeDtypeStruct((ROWS, Z), jnp.float32),
    mesh=_mesh,
    compiler_params=pltpu.CompilerParams(skip_device_barrier=True),
    scratch_types=[
        pltpu.VMEM((GCH, Z), jnp.float32),
        pltpu.VMEM((GCH, Z), jnp.float32),
        pltpu.VMEM((GCH, Z), jnp.float32),
        pltpu.VMEM((RPW // IDXR, IDXR), jnp.int32),
        pltpu.VMEM((CG, Z), jnp.float32),
        pltpu.SemaphoreType.DMA,
        pltpu.SemaphoreType.DMA,
        pltpu.SemaphoreType.DMA,
        pltpu.SemaphoreType.DMA,
    ],
)


def kernel(graph_input, graph_label, W1, b1, W2, b2):
    x = graph_input.reshape(ROWS, Z)
    idx = (graph_label[None, :].astype(jnp.int32)
           + C * jnp.arange(G, dtype=jnp.int32)[:, None]).reshape(ROWS // IDXR, IDXR)
    sums2 = _seg_sum(x, idx)
    lab2d = graph_label.astype(jnp.int32).reshape(CG, IDXR)
    h2 = _dense(sums2, lab2d, W1, b1.reshape(1, 4 * Z), W2, b2.reshape(1, Z))
    out = _gather_add(x, idx, h2)
    return out.reshape(G, N, Z)


# speedup vs baseline: 1.0187x; 1.0187x over previous
import jax
import jax.numpy as jnp
from jax import lax
from jax.experimental import pallas as pl
from jax.experimental.pallas import tpu as pltpu
from jax.experimental.pallas import tpu_sc as plsc

G = 4
N = 4096
Z = 128
C = 8
CG = G * C
NC = 2
NS = 16
NW = NC * NS
ROWS = G * N
RPW = ROWS // NW
SCH = 128
NSCH = RPW // SCH
GCH = 256
LANES = 16
FB = Z // LANES
IDXR = 128

_mesh = plsc.VectorSubcoreMesh(core_axis_name="c", subcore_axis_name="s")


def _seg_sum_body(x_hbm, idx_hbm, out_hbm, xb0, xb1, xb2, xb3, idxbuf, zbuf,
                  acc_sh, sem0, sem1, sem2, sem3, ssem):
    c = lax.axis_index("c")
    s = lax.axis_index("s")
    w = s * NC + c
    base = w * RPW

    xbufs = (xb0, xb1, xb2, xb3)
    sems = (sem0, sem1, sem2, sem3)
    cps = [
        pltpu.async_copy(x_hbm.at[pl.ds(base + j * SCH, SCH)], xbufs[j], sems[j])
        for j in range(NSCH)
    ]
    pltpu.sync_copy(idx_hbm.at[pl.ds(w * (RPW // IDXR), RPW // IDXR)], idxbuf)

    zero = jnp.zeros((LANES,), jnp.float32)
    for r in range(CG):
        for f in range(FB):
            zbuf[r, pl.ds(f * LANES, LANES)] = zero

    @pl.when(s == 0)
    def _():
        pltpu.sync_copy(zbuf, acc_sh)

    plsc.subcore_barrier()

    scs = []
    for j in range(NSCH):
        cps[j].wait()
        scs.append(
            pltpu.async_copy(xbufs[j], acc_sh.at[idxbuf.at[j]], ssem, add=True))
    for sc in scs:
        sc.wait()

    plsc.subcore_barrier()

    @pl.when(s == 0)
    def _():
        pltpu.sync_copy(acc_sh, out_hbm.at[c])


_seg_sum = pl.kernel(
    _seg_sum_body,
    out_type=jax.ShapeDtypeStruct((NC, CG, Z), jnp.float32),
    mesh=_mesh,
    compiler_params=pltpu.CompilerParams(skip_device_barrier=True),
    scratch_types=[
        pltpu.VMEM((SCH, Z), jnp.float32),
        pltpu.VMEM((SCH, Z), jnp.float32),
        pltpu.VMEM((SCH, Z), jnp.float32),
        pltpu.VMEM((SCH, Z), jnp.float32),
        pltpu.VMEM((RPW // IDXR, IDXR), jnp.int32),
        pltpu.VMEM((CG, Z), jnp.float32),
        pltpu.VMEM_SHARED((CG, Z), jnp.float32),
        pltpu.SemaphoreType.DMA,
        pltpu.SemaphoreType.DMA,
        pltpu.SemaphoreType.DMA,
        pltpu.SemaphoreType.DMA,
        pltpu.SemaphoreType.DMA,
    ],
)


def _dense_body(sums_ref, lab_ref, w1_ref, b1_ref, w2_ref, b2_ref, out_ref):
    sums = sums_ref[0] + sums_ref[1]
    lab = lab_ref[...]
    rowc = lax.broadcasted_iota(jnp.int32, (CG, 1), 0) % C
    cnt = jnp.zeros((CG, 1), jnp.float32)
    for cc in range(C):
        n_cc = jnp.sum(jnp.where(lab == cc, 1.0, 0.0))
        cnt = jnp.where(rowc == cc, n_cc, cnt)
    m = sums / jnp.maximum(cnt, 1.0)
    h1 = jnp.dot(m, w1_ref[...], preferred_element_type=jnp.float32)
    h1 = jnp.maximum(h1 + b1_ref[...], 0.0)
    h2 = jnp.dot(h1, w2_ref[...], preferred_element_type=jnp.float32)
    out_ref[...] = h2 + b2_ref[...]


_dense = pl.pallas_call(
    _dense_body,
    out_shape=jax.ShapeDtypeStruct((CG, Z), jnp.float32),
)


def _gadd_chunk(xbuf, idxbuf, h2buf, chunk):
    lo = chunk * (GCH // LANES)

    def body(b, carry):
        jrow = b // (IDXR // LANES)
        goff = b % (IDXR // LANES)
        cv = idxbuf[jrow, pl.ds(goff * LANES, LANES)]
        i0 = (b % (GCH // LANES)) * LANES

        def loads(t):
            cls = cv[t]
            return [h2buf[cls, pl.ds(f * LANES, LANES)] for f in range(FB)]

        vs = loads(0)
        for t in range(LANES):
            nxt = loads(t + 1) if t + 1 < LANES else None
            for f in range(FB):
                plsc.addupdate(xbuf.at[i0 + t, pl.ds(f * LANES, LANES)], vs[f])
            vs = nxt
        return carry

    lax.fori_loop(lo, lo + GCH // LANES, body, 0)


def _gather_add_body(x_hbm, idx_hbm, h2_hbm, out_hbm, xbuf0, xbuf1, idxbuf,
                     h2buf, sem0, sem1, osem):
    c = lax.axis_index("c")
    s = lax.axis_index("s")
    w = s * NC + c
    base = w * RPW

    cp0 = pltpu.async_copy(x_hbm.at[pl.ds(base, GCH)], xbuf0, sem0)
    cp1 = pltpu.async_copy(x_hbm.at[pl.ds(base + GCH, GCH)], xbuf1, sem1)
    pltpu.sync_copy(h2_hbm, h2buf)
    pltpu.sync_copy(idx_hbm.at[pl.ds(w * (RPW // IDXR), RPW // IDXR)], idxbuf)

    cp0.wait()
    _gadd_chunk(xbuf0, idxbuf, h2buf, 0)
    st0 = pltpu.async_copy(xbuf0, out_hbm.at[pl.ds(base, GCH)], osem)
    cp1.wait()
    _gadd_chunk(xbuf1, idxbuf, h2buf, 1)
    st1 = pltpu.async_copy(xbuf1, out_hbm.at[pl.ds(base + GCH, GCH)], osem)
    st0.wait()
    st1.wait()


_gather_add = pl.kernel(
    _gather_add_body,
    out_type=jax.ShapeDtypeStruct((ROWS, Z), jnp.float32),
    mesh=_mesh,
    compiler_params=pltpu.CompilerParams(skip_device_barrier=True),
    scratch_types=[
        pltpu.VMEM((GCH, Z), jnp.float32),
        pltpu.VMEM((GCH, Z), jnp.float32),
        pltpu.VMEM((RPW // IDXR, IDXR), jnp.int32),
        pltpu.VMEM((CG, Z), jnp.float32),
        pltpu.SemaphoreType.DMA,
        pltpu.SemaphoreType.DMA,
        pltpu.SemaphoreType.DMA,
    ],
)


def kernel(graph_input, graph_label, W1, b1, W2, b2):
    x = graph_input.reshape(ROWS, Z)
    idx = (graph_label[None, :].astype(jnp.int32)
           + C * jnp.arange(G, dtype=jnp.int32)[:, None]).reshape(ROWS // IDXR, IDXR)
    sums2 = _seg_sum(x, idx)
    lab2d = graph_label.astype(jnp.int32).reshape(CG, IDXR)
    h2 = _dense(sums2, lab2d, W1, b1.reshape(1, 4 * Z), W2, b2.reshape(1, Z))
    out = _gather_add(x, idx, h2)
    return out.reshape(G, N, Z)
